# baseline (speedup 1.0000x reference)
import functools

import jax
import jax.numpy as jnp
from jax import lax
from jax.experimental import pallas as pl
from jax.experimental.pallas import tpu as pltpu
from jax.experimental.pallas import tpu_sc as plsc

N = 10000
NPAD = 10240
E = 320000
CHUNK = 128
NTILES = 32
NCH = 80
EPT = NCH * CHUNK
EPAD = EPT * NTILES
NROWS = EPAD // CHUNK
D_IN = 128
D_HID = 128
D_OUT = 16
BLK = 2048
DEG_GRP = 8


def _sc_mesh():
    return plsc.VectorSubcoreMesh(core_axis_name="c", subcore_axis_name="s")


_SC_PARAMS = pltpu.CompilerParams(use_tc_tiling_on_sc=False)


@functools.partial(
    pl.kernel,
    out_type=jax.ShapeDtypeStruct((2, NPAD, 1), jnp.float32),
    mesh=_sc_mesh(),
    scratch_types=[
        pltpu.VMEM((NCH, CHUNK), jnp.int32),
        pltpu.VMEM((CHUNK, 1), jnp.float32),
        pltpu.VMEM_SHARED((NPAD, 1), jnp.float32),
        pltpu.SemaphoreType.DMA,
    ],
    compiler_params=_SC_PARAMS,
)
def _deg_pass(dst_hbm, ones_hbm, degp_hbm, didx, ones_v, acc, ssem):
    c = lax.axis_index("c")
    s = lax.axis_index("s")
    wid = c * 16 + s
    rpt = NPAD // 16
    pltpu.sync_copy(ones_hbm.at[pl.ds(s * rpt, rpt)], acc.at[pl.ds(s * rpt, rpt)])
    pltpu.sync_copy(ones_hbm.at[pl.ds(0, CHUNK)], ones_v)
    pltpu.sync_copy(dst_hbm.at[pl.ds(wid * NCH, NCH)], didx)
    plsc.subcore_barrier()

    def body(i, carry):
        descs = []
        for b in range(DEG_GRP):
            j = i * DEG_GRP + b
            descs.append(pltpu.async_copy(ones_v, acc.at[didx.at[j]], ssem, add=True))
        for d in descs:
            d.wait()
        return carry

    lax.fori_loop(0, NCH // DEG_GRP, body, 0)
    plsc.subcore_barrier()
    pltpu.sync_copy(acc.at[pl.ds(s * rpt, rpt)], degp_hbm.at[c, pl.ds(s * rpt, rpt)])


IDXR = 4
NBUF = 4
UNROLL = 4
CH0 = 80
CH1 = 80
HT = 4
CHB = 104
CSP = 72


def _make_agg(halves, W):
    @functools.partial(
        pl.kernel,
        out_type=jax.ShapeDtypeStruct((2, halves, NPAD, W), jnp.float32),
        mesh=_sc_mesh(),
        scratch_types=[
            [pltpu.VMEM((2, CHUNK), jnp.int32)] * IDXR,
            [pltpu.VMEM((CHUNK, W), jnp.float32)] * NBUF,
            pltpu.VMEM_SHARED((NPAD, W), jnp.float32),
            pltpu.VMEM_SHARED((NPAD, W), jnp.float32),
            [pltpu.SemaphoreType.DMA] * IDXR,
            [pltpu.SemaphoreType.DMA] * NBUF,
        ],
        compiler_params=_SC_PARAMS,
    )
    def agg(hs_hbm, idx_hbm, out_hbm, idxr, bufs, rep, acc, isems, gsems):
        c = lax.axis_index("c")
        s = lax.axis_index("s")
        rpt = NPAD // 16
        sl = pl.ds(s * rpt, rpt)
        nch = jnp.where(s < HT, CHB, CSP)
        row0 = (c * (NROWS // 2)
                + jnp.where(s < HT, s * CHB, HT * CHB + (s - HT) * CSP))

        for h in range(halves):
            pltpu.sync_copy(hs_hbm.at[h, sl], rep.at[sl])
            pltpu.sync_copy(hs_hbm.at[h, sl], acc.at[sl])
            for u in range(IDXR):
                pltpu.async_copy(idx_hbm.at[row0 + u], idxr[u], isems[u])
            plsc.subcore_barrier()
            def run_pipe(gsrc, nsteps):
                for u in range(NBUF - 1):
                    pltpu.make_async_copy(idx_hbm.at[row0], idxr[u],
                                          isems[u]).wait()
                    pltpu.async_copy(gsrc.at[idxr[u].at[0]], bufs[u], gsems[u])

                def body(i, carry):
                    for u in range(UNROLL):
                        j = i * UNROLL + u
                        ug = (u + 3) % NBUF
                        pltpu.make_async_copy(idx_hbm.at[row0], idxr[ug],
                                              isems[ug]).wait()
                        pltpu.async_copy(gsrc.at[idxr[ug].at[0]], bufs[ug],
                                         gsems[ug])
                        pltpu.make_async_copy(gsrc.at[idxr[u].at[0]], bufs[u],
                                              gsems[u]).wait()
                        pltpu.sync_copy(bufs[u], acc.at[idxr[u].at[1]],
                                        add=True)
                        nxi = jnp.minimum(j + IDXR, nch - 1)
                        pltpu.async_copy(idx_hbm.at[row0 + nxi], idxr[u],
                                         isems[u])
                    return carry

                lax.fori_loop(0, nsteps, body, 0)
                pltpu.make_async_copy(idx_hbm.at[row0], idxr[3], isems[3]).wait()
                for u in range(NBUF - 1):
                    pltpu.make_async_copy(gsrc.at[idxr[u].at[0]], bufs[u],
                                          gsems[u]).wait()

            @pl.when(s < HT)
            def _():
                run_pipe(hs_hbm.at[h], CHB // UNROLL)

            @pl.when(s >= HT)
            def _():
                run_pipe(rep, CSP // UNROLL)

            plsc.subcore_barrier()
            pltpu.sync_copy(acc.at[sl], out_hbm.at[c, h, sl])

    return agg


_agg128 = _make_agg(2, D_HID // 2)
_agg16 = _make_agg(1, D_OUT)


def _front_body(x_ref, w_ref, degp_ref, dis_ref, hs_ref):
    deg = degp_ref[0] + degp_ref[1] - 1.0
    dis = lax.rsqrt(deg)
    dis_ref[...] = dis
    hs = jnp.dot(x_ref[...], w_ref[...], preferred_element_type=jnp.float32) * dis
    hs_ref[0] = hs[:, :64]
    hs_ref[1] = hs[:, 64:]


def _mid_body(p_ref, hs_ref, dis_ref, b1_ref, w2_ref, o_ref):
    dis = dis_ref[...]
    agg_l = (p_ref[0, 0] + p_ref[1, 0] - hs_ref[0]) * dis + b1_ref[:, :64]
    agg_r = (p_ref[0, 1] + p_ref[1, 1] - hs_ref[1]) * dis + b1_ref[:, 64:]
    z_l = jnp.maximum(agg_l, 0.0)
    z_r = jnp.maximum(agg_r, 0.0)
    o_ref[...] = (jnp.dot(z_l, w2_ref[:64], preferred_element_type=jnp.float32)
                  + jnp.dot(z_r, w2_ref[64:], preferred_element_type=jnp.float32)) * dis


def _final_body(p_ref, hs_ref, dis_ref, b2_ref, o_ref):
    agg = (p_ref[0, 0] + p_ref[1, 0] - hs_ref[...]) * dis_ref[...] + b2_ref[...]
    o_ref[...] = jax.nn.sigmoid(agg)


def _row_spec(d):
    return pl.BlockSpec((BLK, d), lambda i: (i, 0))


def _full_spec(shape):
    nd = len(shape)
    return pl.BlockSpec(shape, lambda i: (0,) * nd)


def _part_spec(halves, d):
    return pl.BlockSpec((2, halves, BLK, d), lambda i: (0, 0, i, 0))


_GRID = NPAD // BLK


def _tc(body, out_d, in_specs):
    return pl.pallas_call(
        body,
        grid=(_GRID,),
        in_specs=in_specs,
        out_specs=_row_spec(out_d),
        out_shape=jax.ShapeDtypeStruct((NPAD, out_d), jnp.float32),
    )


def kernel(x, edge_index, W1, b1, W2, b2):
    ei = edge_index.astype(jnp.int32)
    src = jnp.concatenate([ei[0], jnp.zeros((EPAD - E,), jnp.int32)])
    dst = jnp.concatenate([ei[1], jnp.full((EPAD - E,), N, jnp.int32)])
    src2 = src.reshape(NROWS, CHUNK)
    dst2 = dst.reshape(NROWS, CHUNK)
    idx2 = jnp.stack([src2, dst2], axis=1)
    x_p = jnp.zeros((NPAD, D_IN), jnp.float32).at[:N].set(x)
    ones_col = jnp.ones((NPAD, 1), jnp.float32)
    b1_r = b1.reshape(1, D_HID)
    b2_r = b2.reshape(1, D_OUT)

    degp = _deg_pass(dst2, ones_col)

    dis, hs1 = pl.pallas_call(
        _front_body,
        grid=(_GRID,),
        in_specs=[_row_spec(D_IN), _full_spec((D_IN, D_HID)),
                  pl.BlockSpec((2, BLK, 1), lambda i: (0, i, 0))],
        out_specs=[_row_spec(1),
                   pl.BlockSpec((2, BLK, 64), lambda i: (0, i, 0))],
        out_shape=[jax.ShapeDtypeStruct((NPAD, 1), jnp.float32),
                   jax.ShapeDtypeStruct((2, NPAD, 64), jnp.float32)],
    )(x_p, W1, degp)

    aggp1 = _agg128(hs1, idx2)

    hs2 = _tc(
        _mid_body, D_OUT,
        [_part_spec(2, 64), pl.BlockSpec((2, BLK, 64), lambda i: (0, i, 0)),
         _row_spec(1), _full_spec((1, D_HID)), _full_spec((D_HID, D_OUT))],
    )(aggp1, hs1, dis, b1_r, W2)

    aggp2 = _agg16(hs2.reshape(1, NPAD, D_OUT), idx2)

    fblk = N // _GRID
    out = pl.pallas_call(
        _final_body,
        grid=(_GRID,),
        in_specs=[pl.BlockSpec((2, 1, fblk, D_OUT), lambda i: (0, 0, i, 0)),
                  pl.BlockSpec((fblk, D_OUT), lambda i: (i, 0)),
                  pl.BlockSpec((fblk, 1), lambda i: (i, 0)),
                  _full_spec((1, D_OUT))],
        out_specs=pl.BlockSpec((fblk, D_OUT), lambda i: (i, 0)),
        out_shape=jax.ShapeDtypeStruct((N, D_OUT), jnp.float32),
    )(aggp2, hs2, dis, b2_r)

    return out


# speedup vs baseline: 1.1165x; 1.1165x over previous
import functools

import jax
import jax.numpy as jnp
from jax import lax
from jax.experimental import pallas as pl
from jax.experimental.pallas import tpu as pltpu
from jax.experimental.pallas import tpu_sc as plsc

N = 10000
NPAD = 10240
E = 320000
CHUNK = 128
NTILES = 32
NCH = 80
EPT = NCH * CHUNK
EPAD = EPT * NTILES
NROWS = EPAD // CHUNK
D_IN = 128
D_HID = 128
D_OUT = 16
BLK = 2048
DEG_GRP = 8


def _sc_mesh():
    return plsc.VectorSubcoreMesh(core_axis_name="c", subcore_axis_name="s")


_SC_PARAMS = pltpu.CompilerParams(use_tc_tiling_on_sc=False)


@functools.partial(
    pl.kernel,
    out_type=jax.ShapeDtypeStruct((2, NPAD, 1), jnp.float32),
    mesh=_sc_mesh(),
    scratch_types=[
        pltpu.VMEM((NCH, CHUNK), jnp.int32),
        pltpu.VMEM((CHUNK, 1), jnp.float32),
        pltpu.VMEM_SHARED((NPAD, 1), jnp.float32),
        pltpu.SemaphoreType.DMA,
    ],
    compiler_params=_SC_PARAMS,
)
def _deg_pass(dst_hbm, ones_hbm, degp_hbm, didx, ones_v, acc, ssem):
    c = lax.axis_index("c")
    s = lax.axis_index("s")
    wid = c * 16 + s
    rpt = NPAD // 16
    pltpu.sync_copy(ones_hbm.at[pl.ds(s * rpt, rpt)], acc.at[pl.ds(s * rpt, rpt)])
    pltpu.sync_copy(ones_hbm.at[pl.ds(0, CHUNK)], ones_v)
    pltpu.sync_copy(dst_hbm.at[pl.ds(wid * NCH, NCH)], didx)
    plsc.subcore_barrier()

    def body(i, carry):
        descs = []
        for b in range(DEG_GRP):
            j = i * DEG_GRP + b
            descs.append(pltpu.async_copy(ones_v, acc.at[didx.at[j]], ssem, add=True))
        for d in descs:
            d.wait()
        return carry

    lax.fori_loop(0, NCH // DEG_GRP, body, 0)
    plsc.subcore_barrier()
    pltpu.sync_copy(acc.at[pl.ds(s * rpt, rpt)], degp_hbm.at[c, pl.ds(s * rpt, rpt)])


IDXR = 4
NBUF = 4
UNROLL = 4
CH0 = 80
CH1 = 80


def _make_agg(halves, W):
    @functools.partial(
        pl.kernel,
        out_type=jax.ShapeDtypeStruct((2, halves, NPAD, W), jnp.float32),
        mesh=_sc_mesh(),
        scratch_types=[
            [pltpu.VMEM((2, CHUNK), jnp.int32)] * IDXR,
            [pltpu.VMEM((CHUNK, W), jnp.float32)] * NBUF,
            pltpu.VMEM_SHARED((NPAD, W), jnp.float32),
            pltpu.VMEM_SHARED((NPAD, W), jnp.float32),
            [pltpu.SemaphoreType.DMA] * IDXR,
            [pltpu.SemaphoreType.DMA] * NBUF,
        ],
        compiler_params=_SC_PARAMS,
    )
    def agg(hs_hbm, idx_hbm, out_hbm, idxr, bufs, rep, acc, isems, gsems):
        c = lax.axis_index("c")
        s = lax.axis_index("s")
        rpt = NPAD // 16
        sl = pl.ds(s * rpt, rpt)
        nch = jnp.where(c == 0, CH0, CH1)
        row0 = jnp.where(c == 0, s * CH0, 16 * CH0 + s * CH1)

        for h in range(halves):
            pltpu.sync_copy(hs_hbm.at[h, sl], rep.at[sl])
            pltpu.sync_copy(hs_hbm.at[h, sl], acc.at[sl])
            for u in range(IDXR):
                pltpu.async_copy(idx_hbm.at[row0 + u], idxr[u], isems[u])
            plsc.subcore_barrier()
            for u in range(NBUF - 1):
                pltpu.make_async_copy(idx_hbm.at[row0], idxr[u], isems[u]).wait()
                pltpu.async_copy(rep.at[idxr[u].at[0]], bufs[u], gsems[u])

            def body(i, carry):
                for u in range(UNROLL):
                    j = i * UNROLL + u
                    ug = (u + 3) % NBUF
                    pltpu.make_async_copy(idx_hbm.at[row0], idxr[ug],
                                          isems[ug]).wait()
                    pltpu.async_copy(rep.at[idxr[ug].at[0]], bufs[ug], gsems[ug])
                    pltpu.make_async_copy(rep.at[idxr[u].at[0]], bufs[u],
                                          gsems[u]).wait()
                    pltpu.sync_copy(bufs[u], acc.at[idxr[u].at[1]], add=True)
                    nxi = jnp.minimum(j + IDXR, nch - 1)
                    pltpu.async_copy(idx_hbm.at[row0 + nxi], idxr[u], isems[u])
                return carry

            lax.fori_loop(0, jnp.where(c == 0, CH0 // UNROLL, CH1 // UNROLL),
                          body, 0)
            pltpu.make_async_copy(idx_hbm.at[row0], idxr[3], isems[3]).wait()
            for u in range(NBUF - 1):
                pltpu.make_async_copy(rep.at[idxr[u].at[0]], bufs[u],
                                      gsems[u]).wait()
            plsc.subcore_barrier()
            pltpu.sync_copy(acc.at[sl], out_hbm.at[c, h, sl])

    return agg


_agg128 = _make_agg(2, D_HID // 2)
_agg16 = _make_agg(1, D_OUT)


def _front_body(x_ref, w_ref, degp_ref, dis_ref, hs_ref):
    deg = degp_ref[0] + degp_ref[1] - 1.0
    dis = lax.rsqrt(deg)
    dis_ref[...] = dis
    hs = jnp.dot(x_ref[...], w_ref[...], preferred_element_type=jnp.float32) * dis
    hs_ref[0] = hs[:, :64]
    hs_ref[1] = hs[:, 64:]


def _mid_body(p_ref, hs_ref, dis_ref, b1_ref, w2_ref, o_ref):
    dis = dis_ref[...]
    agg_l = (p_ref[0, 0] + p_ref[1, 0] - hs_ref[0]) * dis + b1_ref[:, :64]
    agg_r = (p_ref[0, 1] + p_ref[1, 1] - hs_ref[1]) * dis + b1_ref[:, 64:]
    z_l = jnp.maximum(agg_l, 0.0)
    z_r = jnp.maximum(agg_r, 0.0)
    o_ref[...] = (jnp.dot(z_l, w2_ref[:64], preferred_element_type=jnp.float32)
                  + jnp.dot(z_r, w2_ref[64:], preferred_element_type=jnp.float32)) * dis


def _final_body(p_ref, hs_ref, dis_ref, b2_ref, o_ref):
    agg = (p_ref[0, 0] + p_ref[1, 0] - hs_ref[...]) * dis_ref[...] + b2_ref[...]
    o_ref[...] = jax.nn.sigmoid(agg)


def _row_spec(d):
    return pl.BlockSpec((BLK, d), lambda i: (i, 0))


def _full_spec(shape):
    nd = len(shape)
    return pl.BlockSpec(shape, lambda i: (0,) * nd)


def _part_spec(halves, d):
    return pl.BlockSpec((2, halves, BLK, d), lambda i: (0, 0, i, 0))


_GRID = NPAD // BLK


def _tc(body, out_d, in_specs):
    return pl.pallas_call(
        body,
        grid=(_GRID,),
        in_specs=in_specs,
        out_specs=_row_spec(out_d),
        out_shape=jax.ShapeDtypeStruct((NPAD, out_d), jnp.float32),
    )


def kernel(x, edge_index, W1, b1, W2, b2):
    ei = edge_index.astype(jnp.int32)
    src = jnp.concatenate([ei[0], jnp.zeros((EPAD - E,), jnp.int32)])
    dst = jnp.concatenate([ei[1], jnp.full((EPAD - E,), N, jnp.int32)])
    src2 = src.reshape(NROWS, CHUNK)
    dst2 = dst.reshape(NROWS, CHUNK)
    idx2 = jnp.stack([src2, dst2], axis=1)
    x_p = jnp.zeros((NPAD, D_IN), jnp.float32).at[:N].set(x)
    ones_col = jnp.ones((NPAD, 1), jnp.float32)
    b1_r = b1.reshape(1, D_HID)
    b2_r = b2.reshape(1, D_OUT)

    degp = _deg_pass(dst2, ones_col)

    dis, hs1 = pl.pallas_call(
        _front_body,
        grid=(_GRID,),
        in_specs=[_row_spec(D_IN), _full_spec((D_IN, D_HID)),
                  pl.BlockSpec((2, BLK, 1), lambda i: (0, i, 0))],
        out_specs=[_row_spec(1),
                   pl.BlockSpec((2, BLK, 64), lambda i: (0, i, 0))],
        out_shape=[jax.ShapeDtypeStruct((NPAD, 1), jnp.float32),
                   jax.ShapeDtypeStruct((2, NPAD, 64), jnp.float32)],
    )(x_p, W1, degp)

    aggp1 = _agg128(hs1, idx2)

    hs2 = _tc(
        _mid_body, D_OUT,
        [_part_spec(2, 64), pl.BlockSpec((2, BLK, 64), lambda i: (0, i, 0)),
         _row_spec(1), _full_spec((1, D_HID)), _full_spec((D_HID, D_OUT))],
    )(aggp1, hs1, dis, b1_r, W2)

    aggp2 = _agg16(hs2.reshape(1, NPAD, D_OUT), idx2)

    fblk = N // _GRID
    out = pl.pallas_call(
        _final_body,
        grid=(_GRID,),
        in_specs=[pl.BlockSpec((2, 1, fblk, D_OUT), lambda i: (0, 0, i, 0)),
                  pl.BlockSpec((fblk, D_OUT), lambda i: (i, 0)),
                  pl.BlockSpec((fblk, 1), lambda i: (i, 0)),
                  _full_spec((1, D_OUT))],
        out_specs=pl.BlockSpec((fblk, D_OUT), lambda i: (i, 0)),
        out_shape=jax.ShapeDtypeStruct((N, D_OUT), jnp.float32),
    )(aggp2, hs2, dis, b2_r)

    return out
